# initial kernel scaffold (unmeasured)
import jax
import jax.numpy as jnp
from jax import lax
from jax.experimental import pallas as pl
from jax.experimental.pallas import tpu as pltpu

N_DEV = 32
N_EXP = 128
E_LOC = 4
T = 1024
D = 512
H = 1024
PACK = D + N_EXP + H


def kernel(x, router_W, route_idx, expert_W):
    def body(x_ref, rw_ref, idx_ref, ew_ref, out_ref,
             comm_ref, send_sems, recv_sems, credit_sem):
        my = lax.axis_index("i")
        left = (my + N_DEV - 1) % N_DEV
        right = (my + 1) % N_DEV

        barrier = pltpu.get_barrier_semaphore()
        for nbr in (left, right):
            pl.semaphore_signal(barrier, inc=1, device_id=(nbr,),
                                device_id_type=pl.DeviceIdType.MESH)
        pl.semaphore_wait(barrier, 2)

        pl.semaphore_signal(credit_sem, inc=1, device_id=(left,),
                            device_id_type=pl.DeviceIdType.MESH)

        def local_contrib(xb, gb, acc):
            r_iota = lax.broadcasted_iota(jnp.int32, (N_EXP, E_LOC), 0)
            c_iota = lax.broadcasted_iota(jnp.int32, (N_EXP, E_LOC), 1)
            sel = (r_iota == my * E_LOC + c_iota).astype(jnp.float32)
            gl = jnp.dot(gb, sel, preferred_element_type=jnp.float32)
            for j in range(E_LOC):
                xs = xb * gl[:, j:j + 1]
                acc = acc + jnp.dot(xs, ew_ref[j],
                                    preferred_element_type=jnp.float32)
            return acc

        xv = x_ref[:, :]
        scores = jnp.dot(xv, rw_ref[:, :], preferred_element_type=jnp.float32)
        m = jnp.max(scores, axis=-1, keepdims=True)
        p = jnp.exp(scores - m)
        p = p / jnp.sum(p, axis=-1, keepdims=True)
        e_iota = lax.broadcasted_iota(jnp.int32, (T, N_EXP), 1)
        mask = ((e_iota == idx_ref[:, 0:1]) | (e_iota == idx_ref[:, 1:2]))
        g = p * mask.astype(jnp.float32)
        g = g / jnp.sum(g, axis=-1, keepdims=True)
        comm_ref[0, :, 0:D] = xv
        comm_ref[0, :, D:D + N_EXP] = g
        comm_ref[0, :, D + N_EXP:PACK] = local_contrib(
            xv, g, jnp.zeros((T, H), jnp.float32))

        def hop(h, s_send, s_recv):
            pl.semaphore_wait(credit_sem, 1)
            rdma = pltpu.make_async_remote_copy(
                src_ref=comm_ref.at[s_send],
                dst_ref=comm_ref.at[s_recv],
                send_sem=send_sems.at[s_send],
                recv_sem=recv_sems.at[s_recv],
                device_id=(right,),
                device_id_type=pl.DeviceIdType.MESH,
            )
            rdma.start()
            rdma.wait()

            @pl.when(h < N_DEV - 1)
            def _():
                pl.semaphore_signal(credit_sem, inc=1, device_id=(left,),
                                    device_id_type=pl.DeviceIdType.MESH)
                xb = comm_ref[s_recv, :, 0:D]
                gb = comm_ref[s_recv, :, D:D + N_EXP]
                acc = comm_ref[s_recv, :, D + N_EXP:PACK]
                comm_ref[s_recv, :, D + N_EXP:PACK] = local_contrib(xb, gb, acc)

        def loop_body(i, carry):
            hop(2 * i, 0, 1)
            hop(2 * i + 1, 1, 0)
            return carry

        lax.fori_loop(0, N_DEV // 2, loop_body, 0)

        out_ref[:, :] = comm_ref[0, :, D + N_EXP:PACK]

    return pl.pallas_call(
        body,
        out_shape=jax.ShapeDtypeStruct((T, H), jnp.float32),
        in_specs=[
            pl.BlockSpec(memory_space=pltpu.VMEM),
            pl.BlockSpec(memory_space=pltpu.VMEM),
            pl.BlockSpec(memory_space=pltpu.VMEM),
            pl.BlockSpec(memory_space=pltpu.VMEM),
        ],
        out_specs=pl.BlockSpec(memory_space=pltpu.VMEM),
        scratch_shapes=[
            pltpu.VMEM((2, T, PACK), jnp.float32),
            pltpu.SemaphoreType.DMA((2,)),
            pltpu.SemaphoreType.DMA((2,)),
            pltpu.SemaphoreType.REGULAR,
        ],
        compiler_params=pltpu.CompilerParams(collective_id=0),
    )(x, router_W, route_idx, expert_W)


# baseline (device time: 2582962 ns/iter reference)
import jax
import jax.numpy as jnp
from jax import lax
from jax.experimental import pallas as pl
from jax.experimental.pallas import tpu as pltpu

N_DEV = 32
N_EXP = 128
E_LOC = 4
T = 1024
D = 512
H = 1024
PACK = D + N_EXP + H


def kernel(x, router_W, route_idx, expert_W):
    def body(x_ref, rw_ref, idx_ref, ew_ref, out_ref,
             comm_ref, send_sems, recv_sems, credit_sem):
        my = lax.axis_index("i")
        left = (my + N_DEV - 1) % N_DEV
        right = (my + 1) % N_DEV

        barrier = pltpu.get_barrier_semaphore()
        for nbr in (left, right):
            pl.semaphore_signal(barrier, inc=1, device_id=(nbr,),
                                device_id_type=pl.DeviceIdType.MESH)
        pl.semaphore_wait(barrier, 2)

        pl.semaphore_signal(credit_sem, inc=1, device_id=(left,),
                            device_id_type=pl.DeviceIdType.MESH)

        def local_contrib(xb, gb, acc):
            r_iota = lax.broadcasted_iota(jnp.int32, (N_EXP, E_LOC), 0)
            c_iota = lax.broadcasted_iota(jnp.int32, (N_EXP, E_LOC), 1)
            sel = (r_iota == my * E_LOC + c_iota).astype(jnp.float32)
            gl = jnp.dot(gb, sel, preferred_element_type=jnp.float32)
            for j in range(E_LOC):
                xs = xb * gl[:, j:j + 1]
                acc = acc + jnp.dot(xs, ew_ref[j],
                                    preferred_element_type=jnp.float32)
            return acc

        xv = x_ref[:, :]
        scores = jnp.dot(xv, rw_ref[:, :], preferred_element_type=jnp.float32)
        m = jnp.max(scores, axis=-1, keepdims=True)
        p = jnp.exp(scores - m)
        p = p / jnp.sum(p, axis=-1, keepdims=True)
        e_iota = lax.broadcasted_iota(jnp.int32, (T, N_EXP), 1)
        mask = ((e_iota == idx_ref[:, 0:1]) | (e_iota == idx_ref[:, 1:2]))
        g = p * mask.astype(jnp.float32)
        g = g / jnp.sum(g, axis=-1, keepdims=True)
        comm_ref[0, :, 0:D] = xv
        comm_ref[0, :, D:D + N_EXP] = g
        comm_ref[0, :, D + N_EXP:PACK] = local_contrib(
            xv, g, jnp.zeros((T, H), jnp.float32))

        def hop(h, s_send, s_recv):
            pl.semaphore_wait(credit_sem, 1)
            rdma = pltpu.make_async_remote_copy(
                src_ref=comm_ref.at[s_send],
                dst_ref=comm_ref.at[s_recv],
                send_sem=send_sems.at[s_send],
                recv_sem=recv_sems.at[s_recv],
                device_id=(right,),
                device_id_type=pl.DeviceIdType.MESH,
            )
            rdma.start()
            rdma.wait()

            @pl.when(h < N_DEV - 1)
            def _():
                pl.semaphore_signal(credit_sem, inc=1, device_id=(left,),
                                    device_id_type=pl.DeviceIdType.MESH)
                xb = comm_ref[s_recv, :, 0:D]
                gb = comm_ref[s_recv, :, D:D + N_EXP]
                acc = comm_ref[s_recv, :, D + N_EXP:PACK]
                comm_ref[s_recv, :, D + N_EXP:PACK] = local_contrib(xb, gb, acc)

        def loop_body(i, carry):
            hop(2 * i, 0, 1)
            hop(2 * i + 1, 1, 0)
            return carry

        lax.fori_loop(0, N_DEV // 2, loop_body, 0)

        out_ref[:, :] = comm_ref[0, :, D + N_EXP:PACK]

    return pl.pallas_call(
        body,
        out_shape=jax.ShapeDtypeStruct((T, H), jnp.float32),
        in_specs=[
            pl.BlockSpec(memory_space=pltpu.VMEM),
            pl.BlockSpec(memory_space=pltpu.VMEM),
            pl.BlockSpec(memory_space=pltpu.VMEM),
            pl.BlockSpec(memory_space=pltpu.VMEM),
        ],
        out_specs=pl.BlockSpec(memory_space=pltpu.VMEM),
        scratch_shapes=[
            pltpu.VMEM((2, T, PACK), jnp.float32),
            pltpu.SemaphoreType.DMA((2,)),
            pltpu.SemaphoreType.DMA((2,)),
            pltpu.SemaphoreType.REGULAR,
        ],
        compiler_params=pltpu.CompilerParams(
            collective_id=0, vmem_limit_bytes=56 * 1024 * 1024),
    )(x, router_W, route_idx, expert_W)


# device time: 1428379 ns/iter; 1.8083x vs baseline; 1.8083x over previous
import jax
import jax.numpy as jnp
from jax import lax
from jax.experimental import pallas as pl
from jax.experimental.pallas import tpu as pltpu

N_DEV = 32
N_EXP = 128
E_LOC = 4
T = 1024
D = 512
H = 1024
XG = D + N_EXP
PACK = XG + H


def kernel(x, router_W, route_idx, expert_W):
    def body(x_ref, rw_ref, idx_ref, ew_ref, out_ref,
             pack_ref, ew_bf_ref,
             xg_send_sems, xg_recv_sems, acc_send_sems, acc_recv_sems,
             credit_sem):
        my = lax.axis_index("i")
        left = (my + N_DEV - 1) % N_DEV
        right = (my + 1) % N_DEV

        barrier = pltpu.get_barrier_semaphore()
        for nbr in (left, right):
            pl.semaphore_signal(barrier, inc=1, device_id=(nbr,),
                                device_id_type=pl.DeviceIdType.MESH)
        pl.semaphore_wait(barrier, 2)

        pl.semaphore_signal(credit_sem, inc=1, device_id=(left,),
                            device_id_type=pl.DeviceIdType.MESH)

        def local_contrib(xb, gb):
            r_iota = lax.broadcasted_iota(jnp.int32, (N_EXP, E_LOC), 0)
            c_iota = lax.broadcasted_iota(jnp.int32, (N_EXP, E_LOC), 1)
            sel = (r_iota == my * E_LOC + c_iota).astype(jnp.bfloat16)
            gl = jnp.dot(gb, sel, preferred_element_type=jnp.float32)
            acc = jnp.zeros((T, H), jnp.float32)
            for j in range(E_LOC):
                d = jnp.dot(xb, ew_bf_ref[j],
                            preferred_element_type=jnp.float32)
                acc = acc + gl[:, j:j + 1] * d
            return acc

        for j in range(E_LOC):
            ew_bf_ref[j] = ew_ref[j].astype(jnp.bfloat16)
        xv = x_ref[:, :]
        scores = jnp.dot(xv, rw_ref[:, :], preferred_element_type=jnp.float32)
        m = jnp.max(scores, axis=-1, keepdims=True)
        p = jnp.exp(scores - m)
        p = p / jnp.sum(p, axis=-1, keepdims=True)
        e_iota = lax.broadcasted_iota(jnp.int32, (T, N_EXP), 1)
        mask = ((e_iota == idx_ref[:, 0:1]) | (e_iota == idx_ref[:, 1:2]))
        g = p * mask.astype(jnp.float32)
        g = g / jnp.sum(g, axis=-1, keepdims=True)
        xb0 = xv.astype(jnp.bfloat16)
        gb0 = g.astype(jnp.bfloat16)
        pack_ref[0, :, 0:D] = xb0
        pack_ref[0, :, D:XG] = gb0
        pack_ref[0, :, XG:PACK] = local_contrib(xb0, gb0).astype(jnp.bfloat16)

        def hop(h, s, r):
            pl.semaphore_wait(credit_sem, 1)
            xg_rdma = pltpu.make_async_remote_copy(
                src_ref=pack_ref.at[s, :, 0:XG],
                dst_ref=pack_ref.at[r, :, 0:XG],
                send_sem=xg_send_sems.at[s],
                recv_sem=xg_recv_sems.at[r],
                device_id=(right,),
                device_id_type=pl.DeviceIdType.MESH,
            )
            acc_rdma = pltpu.make_async_remote_copy(
                src_ref=pack_ref.at[s, :, XG:PACK],
                dst_ref=pack_ref.at[r, :, XG:PACK],
                send_sem=acc_send_sems.at[s],
                recv_sem=acc_recv_sems.at[r],
                device_id=(right,),
                device_id_type=pl.DeviceIdType.MESH,
            )
            xg_rdma.start()
            acc_rdma.start()

            @pl.when(h < N_DEV - 1)
            def _():
                xg_rdma.wait_recv()
                contrib = local_contrib(pack_ref[r, :, 0:D],
                                        pack_ref[r, :, D:XG])
                acc_rdma.wait_recv()
                acc = pack_ref[r, :, XG:PACK].astype(jnp.float32)
                pack_ref[r, :, XG:PACK] = (acc + contrib).astype(jnp.bfloat16)

            @pl.when(h == N_DEV - 1)
            def _():
                xg_rdma.wait_recv()
                acc_rdma.wait_recv()

            xg_rdma.wait_send()
            acc_rdma.wait_send()

            @pl.when(h < N_DEV - 1)
            def _():
                pl.semaphore_signal(credit_sem, inc=1, device_id=(left,),
                                    device_id_type=pl.DeviceIdType.MESH)

        def loop_body(i, carry):
            hop(2 * i, 0, 1)
            hop(2 * i + 1, 1, 0)
            return carry

        lax.fori_loop(0, N_DEV // 2, loop_body, 0)

        out_ref[:, :] = pack_ref[0, :, XG:PACK].astype(jnp.float32)

    return pl.pallas_call(
        body,
        out_shape=jax.ShapeDtypeStruct((T, H), jnp.float32),
        in_specs=[
            pl.BlockSpec(memory_space=pltpu.VMEM),
            pl.BlockSpec(memory_space=pltpu.VMEM),
            pl.BlockSpec(memory_space=pltpu.VMEM),
            pl.BlockSpec(memory_space=pltpu.VMEM),
        ],
        out_specs=pl.BlockSpec(memory_space=pltpu.VMEM),
        scratch_shapes=[
            pltpu.VMEM((2, T, PACK), jnp.bfloat16),
            pltpu.VMEM((E_LOC, D, H), jnp.bfloat16),
            pltpu.SemaphoreType.DMA((2,)),
            pltpu.SemaphoreType.DMA((2,)),
            pltpu.SemaphoreType.DMA((2,)),
            pltpu.SemaphoreType.DMA((2,)),
            pltpu.SemaphoreType.REGULAR,
        ],
        compiler_params=pltpu.CompilerParams(
            collective_id=0, vmem_limit_bytes=56 * 1024 * 1024),
    )(x, router_W, route_idx, expert_W)


# device time: 1214668 ns/iter; 2.1265x vs baseline; 1.1759x over previous
import jax
import jax.numpy as jnp
from jax import lax
from jax.experimental import pallas as pl
from jax.experimental.pallas import tpu as pltpu

N_DEV = 32
N_EXP = 128
E_LOC = 4
T = 1024
D = 512
H = 1024
XG = D + N_EXP
PACK = XG + H


def kernel(x, router_W, route_idx, expert_W):
    def body(x_ref, rw_ref, idx_ref, ew_ref, out_ref,
             pack_ref, ew_bf_ref, contrib_ref,
             xg_send_sems, xg_recv_sems, acc_send_sems, acc_recv_sems,
             xg_credit, acc_credit):
        my = lax.axis_index("i")
        left = (my + N_DEV - 1) % N_DEV
        right = (my + 1) % N_DEV

        barrier = pltpu.get_barrier_semaphore()
        for nbr in (left, right):
            pl.semaphore_signal(barrier, inc=1, device_id=(nbr,),
                                device_id_type=pl.DeviceIdType.MESH)
        pl.semaphore_wait(barrier, 2)

        pl.semaphore_signal(xg_credit, inc=1, device_id=(left,),
                            device_id_type=pl.DeviceIdType.MESH)
        pl.semaphore_signal(acc_credit, inc=1, device_id=(left,),
                            device_id_type=pl.DeviceIdType.MESH)

        def mk_xg(s, r):
            return pltpu.make_async_remote_copy(
                src_ref=pack_ref.at[s, :, 0:XG],
                dst_ref=pack_ref.at[r, :, 0:XG],
                send_sem=xg_send_sems.at[s],
                recv_sem=xg_recv_sems.at[r],
                device_id=(right,),
                device_id_type=pl.DeviceIdType.MESH,
            )

        def mk_acc(s, r):
            return pltpu.make_async_remote_copy(
                src_ref=pack_ref.at[s, :, XG:PACK],
                dst_ref=pack_ref.at[r, :, XG:PACK],
                send_sem=acc_send_sems.at[s],
                recv_sem=acc_recv_sems.at[r],
                device_id=(right,),
                device_id_type=pl.DeviceIdType.MESH,
            )

        def local_contrib(xb, gb):
            r_iota = lax.broadcasted_iota(jnp.int32, (N_EXP, E_LOC), 0)
            c_iota = lax.broadcasted_iota(jnp.int32, (N_EXP, E_LOC), 1)
            sel = (r_iota == my * E_LOC + c_iota).astype(jnp.bfloat16)
            gl = jnp.dot(gb, sel, preferred_element_type=jnp.float32)
            acc = jnp.zeros((T, H), jnp.float32)
            for j in range(E_LOC):
                d = jnp.dot(xb, ew_bf_ref[j],
                            preferred_element_type=jnp.float32)
                acc = acc + gl[:, j:j + 1] * d
            return acc

        for j in range(E_LOC):
            ew_bf_ref[j] = ew_ref[j].astype(jnp.bfloat16)
        xv = x_ref[:, :]
        scores = jnp.dot(xv, rw_ref[:, :], preferred_element_type=jnp.float32)
        m = jnp.max(scores, axis=-1, keepdims=True)
        p = jnp.exp(scores - m)
        p = p / jnp.sum(p, axis=-1, keepdims=True)
        e_iota = lax.broadcasted_iota(jnp.int32, (T, N_EXP), 1)
        mask = ((e_iota == idx_ref[:, 0:1]) | (e_iota == idx_ref[:, 1:2]))
        g = p * mask.astype(jnp.float32)
        g = g / jnp.sum(g, axis=-1, keepdims=True)
        xb0 = xv.astype(jnp.bfloat16)
        gb0 = g.astype(jnp.bfloat16)
        pack_ref[0, :, 0:D] = xb0
        pack_ref[0, :, D:XG] = gb0
        pack_ref[0, :, XG:PACK] = local_contrib(xb0, gb0).astype(jnp.bfloat16)

        def hop(h, s, r):
            @pl.when(h >= 1)
            def _():
                mk_xg(r, s).wait_send()
                pl.semaphore_signal(xg_credit, inc=1, device_id=(left,),
                                    device_id_type=pl.DeviceIdType.MESH)

            @pl.when(h >= 1)
            def _():
                mk_acc(r, s).wait_send()
                pl.semaphore_signal(acc_credit, inc=1, device_id=(left,),
                                    device_id_type=pl.DeviceIdType.MESH)

            pl.semaphore_wait(xg_credit, 1)
            xg = mk_xg(s, r)
            xg.start()

            @pl.when(h >= 1)
            def _():
                mk_acc(r, s).wait_recv()
                acc = pack_ref[s, :, XG:PACK].astype(jnp.float32)
                pack_ref[s, :, XG:PACK] = (
                    acc + contrib_ref[:, :]).astype(jnp.bfloat16)

            pl.semaphore_wait(acc_credit, 1)
            acc_rdma = mk_acc(s, r)
            acc_rdma.start()

            xg.wait_recv()

            @pl.when(h <= N_DEV - 2)
            def _():
                contrib_ref[:, :] = local_contrib(pack_ref[r, :, 0:D],
                                                  pack_ref[r, :, D:XG])

        def loop_body(i, carry):
            hop(2 * i, 0, 1)
            hop(2 * i + 1, 1, 0)
            return carry

        lax.fori_loop(0, N_DEV // 2, loop_body, 0)

        mk_acc(1, 0).wait_recv()
        mk_xg(1, 0).wait_send()
        mk_acc(1, 0).wait_send()
        out_ref[:, :] = pack_ref[0, :, XG:PACK].astype(jnp.float32)

    return pl.pallas_call(
        body,
        out_shape=jax.ShapeDtypeStruct((T, H), jnp.float32),
        in_specs=[
            pl.BlockSpec(memory_space=pltpu.VMEM),
            pl.BlockSpec(memory_space=pltpu.VMEM),
            pl.BlockSpec(memory_space=pltpu.VMEM),
            pl.BlockSpec(memory_space=pltpu.VMEM),
        ],
        out_specs=pl.BlockSpec(memory_space=pltpu.VMEM),
        scratch_shapes=[
            pltpu.VMEM((2, T, PACK), jnp.bfloat16),
            pltpu.VMEM((E_LOC, D, H), jnp.bfloat16),
            pltpu.VMEM((T, H), jnp.float32),
            pltpu.SemaphoreType.DMA((2,)),
            pltpu.SemaphoreType.DMA((2,)),
            pltpu.SemaphoreType.DMA((2,)),
            pltpu.SemaphoreType.DMA((2,)),
            pltpu.SemaphoreType.REGULAR,
            pltpu.SemaphoreType.REGULAR,
        ],
        compiler_params=pltpu.CompilerParams(
            collective_id=0, vmem_limit_bytes=56 * 1024 * 1024),
    )(x, router_W, route_idx, expert_W)


# device time: 1129753 ns/iter; 2.2863x vs baseline; 1.0752x over previous
import jax
import jax.numpy as jnp
from jax import lax
from jax.experimental import pallas as pl
from jax.experimental.pallas import tpu as pltpu

N_DEV = 32
N_EXP = 128
E_LOC = 4
T = 1024
D = 512
H = 1024
PACK = D + H
MROWS = 8


def kernel(x, router_W, route_idx, expert_W):
    def body(x_ref, rw_ref, idx_ref, ew_ref, out_ref,
             pack_ref, meta_ref, ew_bf_ref, contrib_ref,
             xg_send_sems, xg_recv_sems, mt_send_sems, mt_recv_sems,
             acc_send_sems, acc_recv_sems, xg_credit, acc_credit):
        my = lax.axis_index("i")
        left = (my + N_DEV - 1) % N_DEV
        right = (my + 1) % N_DEV

        barrier = pltpu.get_barrier_semaphore()
        for nbr in (left, right):
            pl.semaphore_signal(barrier, inc=1, device_id=(nbr,),
                                device_id_type=pl.DeviceIdType.MESH)
        pl.semaphore_wait(barrier, 2)

        pl.semaphore_signal(xg_credit, inc=1, device_id=(left,),
                            device_id_type=pl.DeviceIdType.MESH)
        pl.semaphore_signal(acc_credit, inc=1, device_id=(left,),
                            device_id_type=pl.DeviceIdType.MESH)

        def mk_xg(s, r):
            return pltpu.make_async_remote_copy(
                src_ref=pack_ref.at[s, :, 0:D],
                dst_ref=pack_ref.at[r, :, 0:D],
                send_sem=xg_send_sems.at[s],
                recv_sem=xg_recv_sems.at[r],
                device_id=(right,),
                device_id_type=pl.DeviceIdType.MESH,
            )

        def mk_mt(s, r):
            return pltpu.make_async_remote_copy(
                src_ref=meta_ref.at[s],
                dst_ref=meta_ref.at[r],
                send_sem=mt_send_sems.at[s],
                recv_sem=mt_recv_sems.at[r],
                device_id=(right,),
                device_id_type=pl.DeviceIdType.MESH,
            )

        def mk_acc(s, r):
            return pltpu.make_async_remote_copy(
                src_ref=pack_ref.at[s, :, D:PACK],
                dst_ref=pack_ref.at[r, :, D:PACK],
                send_sem=acc_send_sems.at[s],
                recv_sem=acc_recv_sems.at[r],
                device_id=(right,),
                device_id_type=pl.DeviceIdType.MESH,
            )

        def local_contrib(xb, meta):
            mt = jnp.transpose(meta, (1, 0)).astype(jnp.float32)
            g0, g1 = mt[:, 0:1], mt[:, 1:2]
            e0, e1 = mt[:, 2:3], mt[:, 3:4]
            acc = jnp.zeros((T, H), jnp.float32)
            for j in range(E_LOC):
                ej = (my * E_LOC + j).astype(jnp.float32)
                glj = (g0 * (e0 == ej).astype(jnp.float32)
                       + g1 * (e1 == ej).astype(jnp.float32))
                d = jnp.dot(xb, ew_bf_ref[j],
                            preferred_element_type=jnp.float32)
                acc = acc + glj * d
            return acc

        for j in range(E_LOC):
            ew_bf_ref[j] = ew_ref[j].astype(jnp.bfloat16)
        xv = x_ref[:, :]
        scores = jnp.dot(xv, rw_ref[:, :], preferred_element_type=jnp.float32)
        mx = jnp.max(scores, axis=-1, keepdims=True)
        p = jnp.exp(scores - mx)
        p = p / jnp.sum(p, axis=-1, keepdims=True)
        e_iota = lax.broadcasted_iota(jnp.int32, (T, N_EXP), 1)
        oh0 = (e_iota == idx_ref[:, 0:1]).astype(jnp.float32)
        oh1 = (e_iota == idx_ref[:, 1:2]).astype(jnp.float32)
        g0 = jnp.sum(p * oh0, axis=-1, keepdims=True)
        g1 = jnp.sum(p * oh1, axis=-1, keepdims=True)
        gs = g0 + g1
        g0, g1 = g0 / gs, g1 / gs
        mcols = jnp.concatenate(
            [g0, g1,
             idx_ref[:, 0:1].astype(jnp.float32),
             idx_ref[:, 1:2].astype(jnp.float32),
             jnp.zeros((T, MROWS - 4), jnp.float32)], axis=1)
        meta0 = jnp.transpose(mcols, (1, 0)).astype(jnp.bfloat16)
        meta_ref[0] = meta0
        xb0 = xv.astype(jnp.bfloat16)
        pack_ref[0, :, 0:D] = xb0
        pack_ref[0, :, D:PACK] = local_contrib(xb0, meta0).astype(jnp.bfloat16)

        def hop(h, s, r):
            @pl.when(h >= 1)
            def _():
                mk_xg(r, s).wait_send()
                mk_mt(r, s).wait_send()
                pl.semaphore_signal(xg_credit, inc=1, device_id=(left,),
                                    device_id_type=pl.DeviceIdType.MESH)

            @pl.when(h >= 1)
            def _():
                mk_acc(r, s).wait_send()
                pl.semaphore_signal(acc_credit, inc=1, device_id=(left,),
                                    device_id_type=pl.DeviceIdType.MESH)

            pl.semaphore_wait(xg_credit, 1)
            xg = mk_xg(s, r)
            mt = mk_mt(s, r)
            xg.start()
            mt.start()

            @pl.when(h >= 1)
            def _():
                mk_acc(r, s).wait_recv()
                acc = pack_ref[s, :, D:PACK].astype(jnp.float32)
                pack_ref[s, :, D:PACK] = (
                    acc + contrib_ref[:, :]).astype(jnp.bfloat16)

            pl.semaphore_wait(acc_credit, 1)
            acc_rdma = mk_acc(s, r)
            acc_rdma.start()

            xg.wait_recv()
            mt.wait_recv()

            @pl.when(h <= N_DEV - 2)
            def _():
                contrib_ref[:, :] = local_contrib(pack_ref[r, :, 0:D],
                                                  meta_ref[r])

        def loop_body(i, carry):
            hop(2 * i, 0, 1)
            hop(2 * i + 1, 1, 0)
            return carry

        lax.fori_loop(0, N_DEV // 2, loop_body, 0)

        mk_acc(1, 0).wait_recv()
        mk_xg(1, 0).wait_send()
        mk_mt(1, 0).wait_send()
        mk_acc(1, 0).wait_send()
        out_ref[:, :] = pack_ref[0, :, D:PACK].astype(jnp.float32)

    return pl.pallas_call(
        body,
        out_shape=jax.ShapeDtypeStruct((T, H), jnp.float32),
        in_specs=[
            pl.BlockSpec(memory_space=pltpu.VMEM),
            pl.BlockSpec(memory_space=pltpu.VMEM),
            pl.BlockSpec(memory_space=pltpu.VMEM),
            pl.BlockSpec(memory_space=pltpu.VMEM),
        ],
        out_specs=pl.BlockSpec(memory_space=pltpu.VMEM),
        scratch_shapes=[
            pltpu.VMEM((2, T, PACK), jnp.bfloat16),
            pltpu.VMEM((2, MROWS, T), jnp.bfloat16),
            pltpu.VMEM((E_LOC, D, H), jnp.bfloat16),
            pltpu.VMEM((T, H), jnp.float32),
            pltpu.SemaphoreType.DMA((2,)),
            pltpu.SemaphoreType.DMA((2,)),
            pltpu.SemaphoreType.DMA((2,)),
            pltpu.SemaphoreType.DMA((2,)),
            pltpu.SemaphoreType.DMA((2,)),
            pltpu.SemaphoreType.DMA((2,)),
            pltpu.SemaphoreType.REGULAR,
            pltpu.SemaphoreType.REGULAR,
        ],
        compiler_params=pltpu.CompilerParams(
            collective_id=0, vmem_limit_bytes=56 * 1024 * 1024),
    )(x, router_W, route_idx, expert_W)
